# Initial kernel scaffold; baseline (speedup 1.0000x reference)
import jax
import jax.numpy as jnp
from jax.experimental import pallas as pl


def kernel(x, edge_index, W1, b1, W2, b2, W3, b3, Wl, bl):
    raise NotImplementedError("write your pallas kernel here")


# speedup vs baseline: 25.8479x; 25.8479x over previous
import functools

import jax
import jax.numpy as jnp
from jax import lax
from jax.experimental import pallas as pl
from jax.experimental.pallas import tpu as pltpu
from jax.experimental.pallas import tpu_sc as plsc

N = 10000
D = 128
E = 320000

NC = 2
NS = 16
NW = NC * NS
EPW = E // NW
C = 40
NBUF = 5
GPW = EPW // (NBUF * C)
NPAD = 10240
RPT = NPAD // NS
ZR = 16

DC = 80
DCH = N // DC


def _mesh():
    return plsc.VectorSubcoreMesh(core_axis_name="c", subcore_axis_name="s")


def _deg_pass(dstd, ones_n):
    @functools.partial(
        pl.kernel,
        out_type=jax.ShapeDtypeStruct((NC, N, 1), jnp.float32),
        mesh=_mesh(),
        scratch_types=[
            pltpu.VMEM((DCH, DC), jnp.int32),
            pltpu.VMEM((DC, 1), jnp.float32),
            pltpu.VMEM_SHARED((N, 1), jnp.float32),
        ],
    )
    def deg_k(dst_hbm, ones_hbm, out_hbm, dst_v, ones_v, acc):
        c = lax.axis_index("c")
        s = lax.axis_index("s")
        wid = c * NS + s
        pltpu.sync_copy(dst_hbm.at[wid], dst_v)
        pltpu.sync_copy(ones_hbm.at[pl.ds(0, DC)], ones_v)
        @pl.when(s == 0)
        def _():
            pltpu.sync_copy(ones_hbm, acc)

        plsc.subcore_barrier()

        def body(j, carry):
            pltpu.sync_copy(ones_v, acc.at[dst_v.at[j]], add=True)
            return carry

        lax.fori_loop(0, DCH, body, 0)
        plsc.subcore_barrier()

        @pl.when(s == 0)
        def _():
            pltpu.sync_copy(acc, out_hbm.at[c])

    return deg_k(dstd, ones_n)


def _edge_pass(g, idx5):
    @functools.partial(
        pl.kernel,
        out_type=jax.ShapeDtypeStruct((NC, NPAD, D), jnp.float32),
        mesh=_mesh(),
        scratch_types=[
            pltpu.VMEM((2, 2, NBUF, C), jnp.int32),
            pltpu.VMEM((NBUF, C, D), jnp.float32),
            pltpu.VMEM((ZR, D), jnp.float32),
            pltpu.VMEM_SHARED((NPAD, D), jnp.float32),
            pltpu.SemaphoreType.DMA((NBUF,)),
            pltpu.SemaphoreType.DMA((2,)),
        ],
    )
    def edge_k(g_hbm, idx_hbm, out_hbm, slab, rows, zbuf, acc, gsem, isem):
        c = lax.axis_index("c")
        s = lax.axis_index("s")
        wid = c * NS + s

        pltpu.sync_copy(idx_hbm.at[wid, 0], slab.at[0])
        pltpu.make_async_copy(idx_hbm.at[wid, 1], slab.at[1], isem.at[1]).start()
        for b in range(NBUF):
            pltpu.make_async_copy(
                g_hbm.at[slab.at[0, 0, b]], rows.at[b], gsem.at[b]
            ).start()

        base = s * RPT

        @pl.when(c == 0)
        def _():
            pltpu.sync_copy(g_hbm.at[pl.ds(base, RPT)], acc.at[pl.ds(base, RPT)])

        @pl.when(c == 1)
        def _():
            def zbody(k, carry):
                zbuf[k // 8, pl.ds((k % 8) * 16, 16)] = jnp.zeros((16,), jnp.float32)
                return carry

            lax.fori_loop(0, ZR * (D // 16), zbody, 0)
            for r in range(RPT // ZR):
                pltpu.sync_copy(zbuf, acc.at[pl.ds(base + r * ZR, ZR)])

        plsc.subcore_barrier()

        def pair(k, carry):
            for slot in range(2):
                gi = 2 * k + slot
                nslot = 1 - slot

                @pl.when(gi < GPW - 1)
                def _():
                    pltpu.make_async_copy(
                        idx_hbm.at[wid, gi + 1], slab.at[nslot], isem.at[nslot]
                    ).wait()

                for b in range(NBUF):
                    pltpu.make_async_copy(
                        g_hbm.at[slab.at[slot, 0, b]], rows.at[b], gsem.at[b]
                    ).wait()
                    pltpu.sync_copy(rows.at[b], acc.at[slab.at[slot, 1, b]], add=True)

                    @pl.when(gi < GPW - 1)
                    def _():
                        pltpu.make_async_copy(
                            g_hbm.at[slab.at[nslot, 0, b]], rows.at[b], gsem.at[b]
                        ).start()

                @pl.when(gi < GPW - 2)
                def _():
                    pltpu.make_async_copy(
                        idx_hbm.at[wid, gi + 2], slab.at[slot], isem.at[slot]
                    ).start()

            return carry

        lax.fori_loop(0, GPW // 2, pair, 0)
        plsc.subcore_barrier()
        pltpu.sync_copy(acc.at[pl.ds(base, RPT)], out_hbm.at[c, pl.ds(base, RPT)])

    return edge_k(g, idx5)


_R = 1000


def _scale_matmul(x2, W, deg0, deg1):
    def body(x_ref, w_ref, d0_ref, d1_ref, g_ref, dinv_ref):
        dinv = lax.rsqrt(d0_ref[...] + d1_ref[...] - 1.0)
        hw = jnp.dot(x_ref[...], w_ref[...], preferred_element_type=jnp.float32)
        g_ref[...] = dinv * hw
        dinv_ref[...] = dinv

    return pl.pallas_call(
        body,
        grid=(N // _R,),
        in_specs=[
            pl.BlockSpec((_R, D), lambda i: (i, 0)),
            pl.BlockSpec((D, D), lambda i: (0, 0)),
            pl.BlockSpec((_R, 1), lambda i: (i, 0)),
            pl.BlockSpec((_R, 1), lambda i: (i, 0)),
        ],
        out_specs=[
            pl.BlockSpec((_R, D), lambda i: (i, 0)),
            pl.BlockSpec((_R, 1), lambda i: (i, 0)),
        ],
        out_shape=[
            jax.ShapeDtypeStruct((NPAD, D), jnp.float32),
            jax.ShapeDtypeStruct((N, 1), jnp.float32),
        ],
    )(x2, W, deg0, deg1)


def _combine_matmul(p, dinv, b, W):
    def body(p_ref, dinv_ref, b_ref, w_ref, g_ref):
        dinv = dinv_ref[...]
        h = jnp.maximum(dinv * (p_ref[0] + p_ref[1]) + b_ref[...], 0.0)
        g_ref[...] = dinv * jnp.dot(h, w_ref[...], preferred_element_type=jnp.float32)

    return pl.pallas_call(
        body,
        grid=(N // _R,),
        in_specs=[
            pl.BlockSpec((NC, _R, D), lambda i: (0, i, 0)),
            pl.BlockSpec((_R, 1), lambda i: (i, 0)),
            pl.BlockSpec((1, D), lambda i: (0, 0)),
            pl.BlockSpec((D, D), lambda i: (0, 0)),
        ],
        out_specs=pl.BlockSpec((_R, D), lambda i: (i, 0)),
        out_shape=jax.ShapeDtypeStruct((NPAD, D), jnp.float32),
    )(p, dinv, b, W)


def _combine_final(p, dinv, b3, Wl, bl):
    def body(p_ref, dinv_ref, b3_ref, wl_ref, bl_ref, o_ref):
        dinv = dinv_ref[...]
        h = jnp.maximum(dinv * (p_ref[0] + p_ref[1]) + b3_ref[...], 0.0)
        o_ref[...] = (
            jnp.dot(h, wl_ref[...], preferred_element_type=jnp.float32) + bl_ref[...]
        )

    return pl.pallas_call(
        body,
        grid=(N // _R,),
        in_specs=[
            pl.BlockSpec((NC, _R, D), lambda i: (0, i, 0)),
            pl.BlockSpec((_R, 1), lambda i: (i, 0)),
            pl.BlockSpec((1, D), lambda i: (0, 0)),
            pl.BlockSpec((D, D), lambda i: (0, 0)),
            pl.BlockSpec((1, D), lambda i: (0, 0)),
        ],
        out_specs=pl.BlockSpec((_R, D), lambda i: (i, 0)),
        out_shape=jax.ShapeDtypeStruct((N, D), jnp.float32),
    )(p, dinv, b3, Wl, bl)


def kernel(x, edge_index, W1, b1, W2, b2, W3, b3, Wl, bl):
    x2 = x.reshape(N, D)
    ei = edge_index.astype(jnp.int32)
    idx5 = ei.reshape(2, NW, GPW, NBUF, C).transpose(1, 2, 0, 3, 4)
    dstd = ei[1].reshape(NW, DCH, DC)
    ones_n = jnp.ones((N, 1), jnp.float32)

    degp = _deg_pass(dstd, ones_n)
    deg0 = degp[0]
    deg1 = degp[1]

    g1, dinv = _scale_matmul(x2, W1, deg0, deg1)
    p1 = _edge_pass(g1, idx5)
    g2 = _combine_matmul(p1, dinv, b1.reshape(1, D), W2)
    p2 = _edge_pass(g2, idx5)
    g3 = _combine_matmul(p2, dinv, b2.reshape(1, D), W3)
    p3 = _edge_pass(g3, idx5)
    out = _combine_final(p3, dinv, b3.reshape(1, D), Wl, bl.reshape(1, D))
    return out.reshape(1, N, D)


# speedup vs baseline: 26.7431x; 1.0346x over previous
import functools

import jax
import jax.numpy as jnp
from jax import lax
from jax.experimental import pallas as pl
from jax.experimental.pallas import tpu as pltpu
from jax.experimental.pallas import tpu_sc as plsc

N = 10000
D = 128
E = 320000

NC = 2
NS = 16
NW = NC * NS
EPW = E // NW
C = 100
NBUF = 2
GPW = EPW // (NBUF * C)
NPAD = 10240
RPT = NPAD // NS
ZR = 16

DC = 80
DCH = N // DC


def _mesh():
    return plsc.VectorSubcoreMesh(core_axis_name="c", subcore_axis_name="s")


def _deg_pass(dstd, ones_n):
    @functools.partial(
        pl.kernel,
        out_type=jax.ShapeDtypeStruct((NC, N, 1), jnp.float32),
        mesh=_mesh(),
        scratch_types=[
            pltpu.VMEM((DCH, DC), jnp.int32),
            pltpu.VMEM((DC, 1), jnp.float32),
            pltpu.VMEM_SHARED((N, 1), jnp.float32),
        ],
    )
    def deg_k(dst_hbm, ones_hbm, out_hbm, dst_v, ones_v, acc):
        c = lax.axis_index("c")
        s = lax.axis_index("s")
        wid = c * NS + s
        pltpu.sync_copy(dst_hbm.at[wid], dst_v)
        pltpu.sync_copy(ones_hbm.at[pl.ds(0, DC)], ones_v)
        @pl.when(s == 0)
        def _():
            pltpu.sync_copy(ones_hbm, acc)

        plsc.subcore_barrier()

        def body(j, carry):
            pltpu.sync_copy(ones_v, acc.at[dst_v.at[j]], add=True)
            return carry

        lax.fori_loop(0, DCH, body, 0)
        plsc.subcore_barrier()

        @pl.when(s == 0)
        def _():
            pltpu.sync_copy(acc, out_hbm.at[c])

    return deg_k(dstd, ones_n)


def _edge_pass(g, idx5):
    @functools.partial(
        pl.kernel,
        out_type=jax.ShapeDtypeStruct((NC, NPAD, D), jnp.float32),
        mesh=_mesh(),
        scratch_types=[
            pltpu.VMEM((2, 2, NBUF, C), jnp.int32),
            pltpu.VMEM((NBUF, C, D), jnp.float32),
            pltpu.VMEM((ZR, D), jnp.float32),
            pltpu.VMEM_SHARED((NPAD, D), jnp.float32),
            pltpu.SemaphoreType.DMA((NBUF,)),
            pltpu.SemaphoreType.DMA((2,)),
        ],
    )
    def edge_k(g_hbm, idx_hbm, out_hbm, slab, rows, zbuf, acc, gsem, isem):
        c = lax.axis_index("c")
        s = lax.axis_index("s")
        wid = c * NS + s

        pltpu.sync_copy(idx_hbm.at[wid, 0], slab.at[0])
        pltpu.make_async_copy(idx_hbm.at[wid, 1], slab.at[1], isem.at[1]).start()
        for b in range(NBUF):
            pltpu.make_async_copy(
                g_hbm.at[slab.at[0, 0, b]], rows.at[b], gsem.at[b]
            ).start()

        base = s * RPT

        @pl.when(c == 0)
        def _():
            pltpu.sync_copy(g_hbm.at[pl.ds(base, RPT)], acc.at[pl.ds(base, RPT)])

        @pl.when(c == 1)
        def _():
            def zbody(k, carry):
                zbuf[k // 8, pl.ds((k % 8) * 16, 16)] = jnp.zeros((16,), jnp.float32)
                return carry

            lax.fori_loop(0, ZR * (D // 16), zbody, 0)
            for r in range(RPT // ZR):
                pltpu.sync_copy(zbuf, acc.at[pl.ds(base + r * ZR, ZR)])

        plsc.subcore_barrier()

        def pair(k, carry):
            for slot in range(2):
                gi = 2 * k + slot
                nslot = 1 - slot

                @pl.when(gi < GPW - 1)
                def _():
                    pltpu.make_async_copy(
                        idx_hbm.at[wid, gi + 1], slab.at[nslot], isem.at[nslot]
                    ).wait()

                for b in range(NBUF):
                    pltpu.make_async_copy(
                        g_hbm.at[slab.at[slot, 0, b]], rows.at[b], gsem.at[b]
                    ).wait()
                    pltpu.sync_copy(rows.at[b], acc.at[slab.at[slot, 1, b]], add=True)

                    @pl.when(gi < GPW - 1)
                    def _():
                        pltpu.make_async_copy(
                            g_hbm.at[slab.at[nslot, 0, b]], rows.at[b], gsem.at[b]
                        ).start()

                @pl.when(gi < GPW - 2)
                def _():
                    pltpu.make_async_copy(
                        idx_hbm.at[wid, gi + 2], slab.at[slot], isem.at[slot]
                    ).start()

            return carry

        lax.fori_loop(0, GPW // 2, pair, 0)
        plsc.subcore_barrier()
        pltpu.sync_copy(acc.at[pl.ds(base, RPT)], out_hbm.at[c, pl.ds(base, RPT)])

    return edge_k(g, idx5)


_R = 1000


def _scale_matmul(x2, W, deg0, deg1):
    def body(x_ref, w_ref, d0_ref, d1_ref, g_ref, dinv_ref):
        dinv = lax.rsqrt(d0_ref[...] + d1_ref[...] - 1.0)
        hw = jnp.dot(x_ref[...], w_ref[...], preferred_element_type=jnp.float32)
        g_ref[...] = dinv * hw
        dinv_ref[...] = dinv

    return pl.pallas_call(
        body,
        grid=(N // _R,),
        in_specs=[
            pl.BlockSpec((_R, D), lambda i: (i, 0)),
            pl.BlockSpec((D, D), lambda i: (0, 0)),
            pl.BlockSpec((_R, 1), lambda i: (i, 0)),
            pl.BlockSpec((_R, 1), lambda i: (i, 0)),
        ],
        out_specs=[
            pl.BlockSpec((_R, D), lambda i: (i, 0)),
            pl.BlockSpec((_R, 1), lambda i: (i, 0)),
        ],
        out_shape=[
            jax.ShapeDtypeStruct((NPAD, D), jnp.float32),
            jax.ShapeDtypeStruct((N, 1), jnp.float32),
        ],
    )(x2, W, deg0, deg1)


def _combine_matmul(p, dinv, b, W):
    def body(p_ref, dinv_ref, b_ref, w_ref, g_ref):
        dinv = dinv_ref[...]
        h = jnp.maximum(dinv * (p_ref[0] + p_ref[1]) + b_ref[...], 0.0)
        g_ref[...] = dinv * jnp.dot(h, w_ref[...], preferred_element_type=jnp.float32)

    return pl.pallas_call(
        body,
        grid=(N // _R,),
        in_specs=[
            pl.BlockSpec((NC, _R, D), lambda i: (0, i, 0)),
            pl.BlockSpec((_R, 1), lambda i: (i, 0)),
            pl.BlockSpec((1, D), lambda i: (0, 0)),
            pl.BlockSpec((D, D), lambda i: (0, 0)),
        ],
        out_specs=pl.BlockSpec((_R, D), lambda i: (i, 0)),
        out_shape=jax.ShapeDtypeStruct((NPAD, D), jnp.float32),
    )(p, dinv, b, W)


def _combine_final(p, dinv, b3, Wl, bl):
    def body(p_ref, dinv_ref, b3_ref, wl_ref, bl_ref, o_ref):
        dinv = dinv_ref[...]
        h = jnp.maximum(dinv * (p_ref[0] + p_ref[1]) + b3_ref[...], 0.0)
        o_ref[...] = (
            jnp.dot(h, wl_ref[...], preferred_element_type=jnp.float32) + bl_ref[...]
        )

    return pl.pallas_call(
        body,
        grid=(N // _R,),
        in_specs=[
            pl.BlockSpec((NC, _R, D), lambda i: (0, i, 0)),
            pl.BlockSpec((_R, 1), lambda i: (i, 0)),
            pl.BlockSpec((1, D), lambda i: (0, 0)),
            pl.BlockSpec((D, D), lambda i: (0, 0)),
            pl.BlockSpec((1, D), lambda i: (0, 0)),
        ],
        out_specs=pl.BlockSpec((_R, D), lambda i: (i, 0)),
        out_shape=jax.ShapeDtypeStruct((N, D), jnp.float32),
    )(p, dinv, b3, Wl, bl)


def kernel(x, edge_index, W1, b1, W2, b2, W3, b3, Wl, bl):
    x2 = x.reshape(N, D)
    ei = edge_index.astype(jnp.int32)
    idx5 = ei.reshape(2, NW, GPW, NBUF, C).transpose(1, 2, 0, 3, 4)
    dstd = ei[1].reshape(NW, DCH, DC)
    ones_n = jnp.ones((N, 1), jnp.float32)

    degp = _deg_pass(dstd, ones_n)
    deg0 = degp[0]
    deg1 = degp[1]

    g1, dinv = _scale_matmul(x2, W1, deg0, deg1)
    p1 = _edge_pass(g1, idx5)
    g2 = _combine_matmul(p1, dinv, b1.reshape(1, D), W2)
    p2 = _edge_pass(g2, idx5)
    g3 = _combine_matmul(p2, dinv, b2.reshape(1, D), W3)
    p3 = _edge_pass(g3, idx5)
    out = _combine_final(p3, dinv, b3.reshape(1, D), Wl, bl.reshape(1, D))
    return out.reshape(1, N, D)


# speedup vs baseline: 27.7671x; 1.0383x over previous
import functools

import jax
import jax.numpy as jnp
from jax import lax
from jax.experimental import pallas as pl
from jax.experimental.pallas import tpu as pltpu
from jax.experimental.pallas import tpu_sc as plsc

N = 10000
D = 128
E = 320000

NC = 2
NS = 16
NW = NC * NS
EPW = E // NW
C = 100
NBUF = 2
GPW = EPW // (NBUF * C)
NPAD = 10240
RPT = NPAD // NS
ZR = 16

DC = 80
DCH = N // DC


def _mesh():
    return plsc.VectorSubcoreMesh(core_axis_name="c", subcore_axis_name="s")


DRING = 5


def _deg_pass(dstd, ones_n):
    @functools.partial(
        pl.kernel,
        out_type=jax.ShapeDtypeStruct((NC, N, 1), jnp.float32),
        mesh=_mesh(),
        scratch_types=[
            pltpu.VMEM((DCH, DC), jnp.int32),
            pltpu.VMEM((DC, 1), jnp.float32),
            pltpu.VMEM_SHARED((N, 1), jnp.float32),
            pltpu.SemaphoreType.DMA((DRING,)),
        ],
    )
    def deg_k(dst_hbm, ones_hbm, out_hbm, dst_v, ones_v, acc, dsem):
        c = lax.axis_index("c")
        s = lax.axis_index("s")
        wid = c * NS + s
        pltpu.sync_copy(dst_hbm.at[wid], dst_v)
        pltpu.sync_copy(ones_hbm.at[pl.ds(0, DC)], ones_v)
        @pl.when(s == 0)
        def _():
            pltpu.sync_copy(ones_hbm, acc)

        plsc.subcore_barrier()

        def body(k, carry):
            for r in range(DRING):
                j = DRING * k + r

                @pl.when(k > 0)
                def _():
                    pltpu.make_async_copy(
                        ones_v, acc.at[dst_v.at[j - DRING]], dsem.at[r]
                    ).wait()

                pltpu.async_copy(ones_v, acc.at[dst_v.at[j]], dsem.at[r], add=True)
            return carry

        lax.fori_loop(0, DCH // DRING, body, 0)
        for r in range(DRING):
            pltpu.make_async_copy(
                ones_v, acc.at[dst_v.at[DCH - DRING + r]], dsem.at[r]
            ).wait()
        plsc.subcore_barrier()

        @pl.when(s == 0)
        def _():
            pltpu.sync_copy(acc, out_hbm.at[c])

    return deg_k(dstd, ones_n)


def _edge_pass(g, srcx, dstx):
    @functools.partial(
        pl.kernel,
        out_type=jax.ShapeDtypeStruct((NC, NPAD, D), jnp.float32),
        mesh=_mesh(),
        scratch_types=[
            pltpu.VMEM((2, NBUF, C), jnp.int32),
            pltpu.VMEM((2, NBUF, C), jnp.int32),
            pltpu.VMEM((NBUF, C, D), jnp.float32),
            pltpu.VMEM((ZR, D), jnp.float32),
            pltpu.VMEM_SHARED((NPAD, D), jnp.float32),
            pltpu.SemaphoreType.DMA((NBUF,)),
            pltpu.SemaphoreType.DMA((2,)),
        ],
    )
    def edge_k(g_hbm, src_hbm, dst_hbm, out_hbm, sslab, dslab, rows, zbuf, acc, gsem, isem):
        c = lax.axis_index("c")
        s = lax.axis_index("s")
        wid = c * NS + s

        pltpu.sync_copy(src_hbm.at[wid, 0], sslab.at[0])
        pltpu.sync_copy(dst_hbm.at[wid, 0], dslab.at[0])
        pltpu.make_async_copy(src_hbm.at[wid, 1], sslab.at[1], isem.at[1]).start()
        pltpu.make_async_copy(dst_hbm.at[wid, 1], dslab.at[1], isem.at[1]).start()
        for b in range(NBUF):
            pltpu.make_async_copy(
                g_hbm.at[sslab.at[0, b]], rows.at[b], gsem.at[b]
            ).start()

        base = s * RPT

        @pl.when(c == 0)
        def _():
            pltpu.sync_copy(g_hbm.at[pl.ds(base, RPT)], acc.at[pl.ds(base, RPT)])

        @pl.when(c == 1)
        def _():
            def zbody(k, carry):
                zbuf[k // 8, pl.ds((k % 8) * 16, 16)] = jnp.zeros((16,), jnp.float32)
                return carry

            lax.fori_loop(0, ZR * (D // 16), zbody, 0)
            for r in range(RPT // ZR):
                pltpu.sync_copy(zbuf, acc.at[pl.ds(base + r * ZR, ZR)])

        plsc.subcore_barrier()

        def pair(k, carry):
            for slot in range(2):
                gi = 2 * k + slot
                nslot = 1 - slot

                @pl.when(gi < GPW - 1)
                def _():
                    pltpu.make_async_copy(
                        src_hbm.at[wid, gi + 1], sslab.at[nslot], isem.at[nslot]
                    ).wait()
                    pltpu.make_async_copy(
                        dst_hbm.at[wid, gi + 1], dslab.at[nslot], isem.at[nslot]
                    ).wait()

                for b in range(NBUF):
                    pltpu.make_async_copy(
                        g_hbm.at[sslab.at[slot, b]], rows.at[b], gsem.at[b]
                    ).wait()
                    pltpu.sync_copy(rows.at[b], acc.at[dslab.at[slot, b]], add=True)

                    @pl.when(gi < GPW - 1)
                    def _():
                        pltpu.make_async_copy(
                            g_hbm.at[sslab.at[nslot, b]], rows.at[b], gsem.at[b]
                        ).start()

                @pl.when(gi < GPW - 2)
                def _():
                    pltpu.make_async_copy(
                        src_hbm.at[wid, gi + 2], sslab.at[slot], isem.at[slot]
                    ).start()
                    pltpu.make_async_copy(
                        dst_hbm.at[wid, gi + 2], dslab.at[slot], isem.at[slot]
                    ).start()

            return carry

        lax.fori_loop(0, GPW // 2, pair, 0)
        plsc.subcore_barrier()
        pltpu.sync_copy(acc.at[pl.ds(base, RPT)], out_hbm.at[c, pl.ds(base, RPT)])

    return edge_k(g, srcx, dstx)


_R = 1000


def _scale_matmul(x, W, degp):
    def body(x_ref, w_ref, dp_ref, g_ref, dinv_ref):
        dinv = lax.rsqrt(dp_ref[0] + dp_ref[1] - 1.0)
        hw = jnp.dot(x_ref[0], w_ref[...], preferred_element_type=jnp.float32)
        g_ref[...] = dinv * hw
        dinv_ref[...] = dinv

    return pl.pallas_call(
        body,
        grid=(N // _R,),
        in_specs=[
            pl.BlockSpec((1, _R, D), lambda i: (0, i, 0)),
            pl.BlockSpec((D, D), lambda i: (0, 0)),
            pl.BlockSpec((NC, _R, 1), lambda i: (0, i, 0)),
        ],
        out_specs=[
            pl.BlockSpec((_R, D), lambda i: (i, 0)),
            pl.BlockSpec((_R, 1), lambda i: (i, 0)),
        ],
        out_shape=[
            jax.ShapeDtypeStruct((NPAD, D), jnp.float32),
            jax.ShapeDtypeStruct((N, 1), jnp.float32),
        ],
    )(x, W, degp)


def _combine_matmul(p, dinv, b, W):
    def body(p_ref, dinv_ref, b_ref, w_ref, g_ref):
        dinv = dinv_ref[...]
        h = jnp.maximum(dinv * (p_ref[0] + p_ref[1]) + b_ref[...], 0.0)
        g_ref[...] = dinv * jnp.dot(h, w_ref[...], preferred_element_type=jnp.float32)

    return pl.pallas_call(
        body,
        grid=(N // _R,),
        in_specs=[
            pl.BlockSpec((NC, _R, D), lambda i: (0, i, 0)),
            pl.BlockSpec((_R, 1), lambda i: (i, 0)),
            pl.BlockSpec((1, D), lambda i: (0, 0)),
            pl.BlockSpec((D, D), lambda i: (0, 0)),
        ],
        out_specs=pl.BlockSpec((_R, D), lambda i: (i, 0)),
        out_shape=jax.ShapeDtypeStruct((NPAD, D), jnp.float32),
    )(p, dinv, b, W)


def _combine_final(p, dinv, b3, Wl, bl):
    def body(p_ref, dinv_ref, b3_ref, wl_ref, bl_ref, o_ref):
        dinv = dinv_ref[...]
        h = jnp.maximum(dinv * (p_ref[0] + p_ref[1]) + b3_ref[...], 0.0)
        o_ref[...] = (
            jnp.dot(h, wl_ref[...], preferred_element_type=jnp.float32) + bl_ref[...]
        )

    return pl.pallas_call(
        body,
        grid=(N // _R,),
        in_specs=[
            pl.BlockSpec((NC, _R, D), lambda i: (0, i, 0)),
            pl.BlockSpec((_R, 1), lambda i: (i, 0)),
            pl.BlockSpec((1, D), lambda i: (0, 0)),
            pl.BlockSpec((D, D), lambda i: (0, 0)),
            pl.BlockSpec((1, D), lambda i: (0, 0)),
        ],
        out_specs=pl.BlockSpec((_R, D), lambda i: (i, 0)),
        out_shape=jax.ShapeDtypeStruct((N, D), jnp.float32),
    )(p, dinv, b3, Wl, bl)


def kernel(x, edge_index, W1, b1, W2, b2, W3, b3, Wl, bl):
    ei = edge_index.astype(jnp.int32)
    srcx = ei[0].reshape(NW, GPW, NBUF, C)
    dstx = ei[1].reshape(NW, GPW, NBUF, C)
    dstd = ei[1].reshape(NW, DCH, DC)
    ones_n = jnp.ones((N, 1), jnp.float32)

    degp = _deg_pass(dstd, ones_n)

    g1, dinv = _scale_matmul(x, W1, degp)
    p1 = _edge_pass(g1, srcx, dstx)
    g2 = _combine_matmul(p1, dinv, b1.reshape(1, D), W2)
    p2 = _edge_pass(g2, srcx, dstx)
    g3 = _combine_matmul(p2, dinv, b2.reshape(1, D), W3)
    p3 = _edge_pass(g3, srcx, dstx)
    out = _combine_final(p3, dinv, b3.reshape(1, D), Wl, bl.reshape(1, D))
    return out.reshape(1, N, D)
